# baseline (speedup 1.0000x reference)
import jax
import jax.numpy as jnp
from jax.experimental import pallas as pl

N_CODES = 1024
CODE_DIM = 64
ROWS = 16384
BLK = 1024


def _vq_block(xt_ref, cb_ref, cbt2_ref, en_ref, o_ref):
    x = jnp.transpose(xt_ref[0], (1, 0))
    xn = jnp.sum(x ** 2, axis=1, keepdims=True)
    mm2 = jnp.dot(x, cbt2_ref[...])
    d = xn + en_ref[...] + mm2
    m = jnp.min(d, axis=1, keepdims=True)
    k_iota = jax.lax.broadcasted_iota(jnp.int32, d.shape, 1)
    idx = jnp.min(jnp.where(d == m, k_iota, N_CODES), axis=1, keepdims=True)
    oh = (idx == k_iota).astype(jnp.float32)
    o_ref[...] = jnp.dot(oh, cb_ref[...])


def kernel(vectors, codebook):
    xt = vectors.reshape(vectors.shape[0], CODE_DIM, -1)
    cbt2 = -2.0 * codebook.T
    en = jnp.sum(codebook ** 2, axis=1)[None, :]
    out = pl.pallas_call(
        _vq_block,
        grid=(ROWS // BLK,),
        in_specs=[
            pl.BlockSpec((1, CODE_DIM, BLK), lambda i: (i, 0, 0)),
            pl.BlockSpec((N_CODES, CODE_DIM), lambda i: (0, 0)),
            pl.BlockSpec((CODE_DIM, N_CODES), lambda i: (0, 0)),
            pl.BlockSpec((1, N_CODES), lambda i: (0, 0)),
        ],
        out_specs=pl.BlockSpec((BLK, CODE_DIM), lambda i: (i, 0)),
        out_shape=jax.ShapeDtypeStruct((ROWS, CODE_DIM), jnp.float32),
    )(xt, codebook, cbt2, en)
    return out.reshape(vectors.shape[0], 32, 32, CODE_DIM)
